# baseline (speedup 1.0000x reference)
import functools

import jax
import jax.numpy as jnp
from jax import lax
from jax.experimental import pallas as pl
from jax.experimental.pallas import tpu as pltpu
from jax.experimental.pallas import tpu_sc as plsc

HP = 128
HPK = 64
CP = 64


def _gw_body(t_ref, w_ref, b_ref, o_ref):
    o_ref[...] = (
        jnp.dot(t_ref[...], w_ref[...], preferred_element_type=jnp.float32)
        + b_ref[...]
    )


def _table_times_w(table, Wp, bp, rblk=2000):
    V, E = table.shape
    return pl.pallas_call(
        _gw_body,
        grid=(V // rblk,),
        in_specs=[
            pl.BlockSpec((rblk, E), lambda i: (i, 0)),
            pl.BlockSpec((E, HP), lambda i: (0, 0)),
            pl.BlockSpec((1, HP), lambda i: (0, 0)),
        ],
        out_specs=pl.BlockSpec((rblk, HP), lambda i: (i, 0)),
        out_shape=jax.ShapeDtypeStruct((V, HP), jnp.float32),
    )(table, Wp, bp)


def _make_gather(LB, nc, ns, k_grp=5):
    nw = nc * ns
    per_w = LB // nw
    grp = k_grp * 128
    n_groups = per_w // grp
    assert per_w % grp == 0

    mesh = plsc.VectorSubcoreMesh(core_axis_name="c", subcore_axis_name="s")

    @functools.partial(
        pl.kernel,
        mesh=mesh,
        out_type=jax.ShapeDtypeStruct((LB, HP), jnp.float32),
        scratch_types=[
            pltpu.VMEM((per_w // 128, 128), jnp.int32),
            pltpu.VMEM((grp, HP), jnp.float32),
            pltpu.SemaphoreType.DMA,
        ],
    )
    def gather_k(g_hbm, idx_hbm, out_hbm, idx_v, rows_v, sem):
        wid = lax.axis_index("s") * nc + lax.axis_index("c")
        base = wid * per_w
        pltpu.sync_copy(idx_hbm.at[wid], idx_v)
        for g in range(n_groups):
            off = base + g * grp
            descs = [
                pltpu.async_copy(
                    g_hbm.at[idx_v.at[g * k_grp + j]],
                    rows_v.at[pl.ds(j * 128, 128)],
                    sem,
                )
                for j in range(k_grp)
            ]
            for d in descs:
                d.wait()
            pltpu.sync_copy(rows_v, out_hbm.at[pl.ds(off, grp)])

    return gather_k


def _scan_body(n_steps, t_blk, xb_ref, idx_ref, u_ref, wfc_ref, bfc_ref,
               o_ref, h_ref):
    step = pl.program_id(0)

    @pl.when(step == 0)
    def _init():
        h_ref[...] = jnp.zeros_like(h_ref)

    HB = xb_ref.shape[0] // 2
    ha = h_ref[: HB, :]
    hb = h_ref[HB:, :]
    u = u_ref[...]
    for t in range(t_blk):
        xa = xb_ref[:HB, t, :]
        xb = xb_ref[HB:, t, :]
        ma = idx_ref[:HB, t, :] != 0
        mb = idx_ref[HB:, t, :] != 0
        hna = jnp.tanh(xa + jnp.dot(ha, u, preferred_element_type=jnp.float32))
        hnb = jnp.tanh(xb + jnp.dot(hb, u, preferred_element_type=jnp.float32))
        ha = jnp.where(ma, hna, ha)
        hb = jnp.where(mb, hnb, hb)
    h_ref[: HB, :] = ha
    h_ref[HB:, :] = hb

    @pl.when(step == n_steps - 1)
    def _head():
        h = jnp.concatenate([ha, hb], axis=0)
        logits = (
            jnp.dot(h, wfc_ref[...], preferred_element_type=jnp.float32)
            + bfc_ref[...]
        )
        mx = jnp.max(logits, axis=-1, keepdims=True)
        e = jnp.exp(logits - mx)
        o_ref[...] = e / jnp.sum(e, axis=-1, keepdims=True)


def _rnn_scan(xb3, idx3, Up, Wfcp, bfcp, t_blk=8):
    B, L, _ = xb3.shape
    n_steps = L // t_blk
    return pl.pallas_call(
        functools.partial(_scan_body, n_steps, t_blk),
        grid=(n_steps,),
        in_specs=[
            pl.BlockSpec((B, t_blk, HP), lambda i: (0, i, 0)),
            pl.BlockSpec((B, t_blk, 1), lambda i: (0, i, 0)),
            pl.BlockSpec((HP, HP), lambda i: (0, 0)),
            pl.BlockSpec((HP, CP), lambda i: (0, 0)),
            pl.BlockSpec((1, CP), lambda i: (0, 0)),
        ],
        out_specs=pl.BlockSpec((B, CP), lambda i: (0, 0)),
        out_shape=jax.ShapeDtypeStruct((B, CP), jnp.float32),
        scratch_shapes=[pltpu.VMEM((B, HP), jnp.float32)],
    )(xb3, idx3, Up, Wfcp, bfcp)


def kernel(indices, table, W, U, b, Wfc, bfc):
    B, L = indices.shape
    V, E = table.shape
    H = W.shape[1]
    C = Wfc.shape[1]

    Wp = jnp.pad(W, ((0, 0), (0, HP - H)))
    bp = jnp.pad(b, (0, HP - H)).reshape(1, HP)
    Up = jnp.pad(U, ((0, HP - H), (0, HP - H)))
    Wfcp = jnp.pad(Wfc, ((0, HP - H), (0, CP - C)))
    bfcp = jnp.pad(bfc, (0, CP - C), constant_values=-1e30).reshape(1, CP)

    G = _table_times_w(table, Wp, bp)

    LB = L * B
    info = plsc.get_sparse_core_info()
    nw = info.num_cores * info.num_subcores
    idx2 = indices.reshape(nw, LB // nw // 128, 128).astype(jnp.int32)
    gather_k = _make_gather(LB, info.num_cores, info.num_subcores)
    xb = gather_k(G, idx2)

    xb3 = xb.reshape(B, L, HP)
    idx3 = indices.reshape(B, L, 1)
    probs = _rnn_scan(xb3, idx3, Up, Wfcp, bfcp)
    return probs[:, :C]


# speedup vs baseline: 1.3784x; 1.3784x over previous
import functools

import jax
import jax.numpy as jnp
from jax import lax
from jax.experimental import pallas as pl
from jax.experimental.pallas import tpu as pltpu
from jax.experimental.pallas import tpu_sc as plsc

HP = 128
HPK = 64
CP = 64


def _gw_body(tt_ref, w_ref, b_ref, o_ref):
    o_ref[...] = (
        lax.dot_general(
            tt_ref[...], w_ref[...],
            dimension_numbers=(((0,), (0,)), ((), ())),
            preferred_element_type=jnp.float32,
        )
        + b_ref[...]
    )


def _table_times_w(tableT, Wp, bp, vblk=2048):
    E, V = tableT.shape
    return pl.pallas_call(
        _gw_body,
        grid=(pl.cdiv(V, vblk),),
        in_specs=[
            pl.BlockSpec((E, vblk), lambda i: (0, i)),
            pl.BlockSpec((E, HP), lambda i: (0, 0)),
            pl.BlockSpec((1, HP), lambda i: (0, 0)),
        ],
        out_specs=pl.BlockSpec((vblk, HP), lambda i: (i, 0)),
        out_shape=jax.ShapeDtypeStruct((V, HP), jnp.float32),
    )(tableT, Wp, bp)


def _make_gather(LB, nc, ns, k_grp=5):
    nw = nc * ns
    per_w = LB // nw
    grp = k_grp * 128
    n_groups = per_w // grp
    assert per_w % grp == 0

    mesh = plsc.VectorSubcoreMesh(core_axis_name="c", subcore_axis_name="s")

    @functools.partial(
        pl.kernel,
        mesh=mesh,
        out_type=jax.ShapeDtypeStruct((LB, HP), jnp.float32),
        scratch_types=[
            pltpu.VMEM((per_w // 128, 128), jnp.int32),
            pltpu.VMEM((grp, HP), jnp.float32),
            pltpu.SemaphoreType.DMA,
        ],
    )
    def gather_k(g_hbm, idx_hbm, out_hbm, idx_v, rows_v, sem):
        wid = lax.axis_index("s") * nc + lax.axis_index("c")
        base = wid * per_w
        pltpu.sync_copy(idx_hbm.at[wid], idx_v)
        for g in range(n_groups):
            off = base + g * grp
            descs = [
                pltpu.async_copy(
                    g_hbm.at[idx_v.at[g * k_grp + j]],
                    rows_v.at[pl.ds(j * 128, 128)],
                    sem,
                )
                for j in range(k_grp)
            ]
            for d in descs:
                d.wait()
            pltpu.sync_copy(rows_v, out_hbm.at[pl.ds(off, grp)])

    return gather_k


def _scan_body(n_steps, t_blk, xb_ref, idx_ref, u_ref, wfc_ref, bfc_ref,
               o_ref, h_ref):
    step = pl.program_id(0)

    @pl.when(step == 0)
    def _init():
        h_ref[...] = jnp.zeros_like(h_ref)

    HB = xb_ref.shape[1] // 2
    ha = h_ref[: HB, :]
    hb = h_ref[HB:, :]
    u = u_ref[...]
    for t in range(t_blk):
        xa = xb_ref[t, :HB, :]
        xb = xb_ref[t, HB:, :]
        ma = idx_ref[:HB, t, :] != 0
        mb = idx_ref[HB:, t, :] != 0
        hna = jnp.tanh(xa + jnp.dot(ha, u, preferred_element_type=jnp.float32))
        hnb = jnp.tanh(xb + jnp.dot(hb, u, preferred_element_type=jnp.float32))
        ha = jnp.where(ma, hna, ha)
        hb = jnp.where(mb, hnb, hb)
    h_ref[: HB, :] = ha
    h_ref[HB:, :] = hb

    @pl.when(step == n_steps - 1)
    def _head():
        h = jnp.concatenate([ha, hb], axis=0)
        logits = (
            jnp.dot(h, wfc_ref[...], preferred_element_type=jnp.float32)
            + bfc_ref[...]
        )
        mx = jnp.max(logits, axis=-1, keepdims=True)
        e = jnp.exp(logits - mx)
        o_ref[...] = e / jnp.sum(e, axis=-1, keepdims=True)


def _rnn_scan(xb3, idx3, Up, Wfcp, bfcp, t_blk=8):
    L, B, _ = xb3.shape
    n_steps = L // t_blk
    return pl.pallas_call(
        functools.partial(_scan_body, n_steps, t_blk),
        grid=(n_steps,),
        in_specs=[
            pl.BlockSpec((t_blk, B, HP), lambda i: (i, 0, 0)),
            pl.BlockSpec((B, t_blk, 1), lambda i: (0, i, 0)),
            pl.BlockSpec((HP, HP), lambda i: (0, 0)),
            pl.BlockSpec((HP, CP), lambda i: (0, 0)),
            pl.BlockSpec((1, CP), lambda i: (0, 0)),
        ],
        out_specs=pl.BlockSpec((B, CP), lambda i: (0, 0)),
        out_shape=jax.ShapeDtypeStruct((B, CP), jnp.float32),
        scratch_shapes=[pltpu.VMEM((B, HP), jnp.float32)],
    )(xb3, idx3, Up, Wfcp, bfcp)


def kernel(indices, table, W, U, b, Wfc, bfc):
    B, L = indices.shape
    V, E = table.shape
    H = W.shape[1]
    C = Wfc.shape[1]

    Wp = jnp.pad(W, ((0, 0), (0, HP - H)))
    bp = jnp.pad(b, (0, HP - H)).reshape(1, HP)
    Up = jnp.pad(U, ((0, HP - H), (0, HP - H)))
    Wfcp = jnp.pad(Wfc, ((0, HP - H), (0, CP - C)))
    bfcp = jnp.pad(bfc, (0, CP - C), constant_values=-1e30).reshape(1, CP)

    G = _table_times_w(jnp.swapaxes(table, 0, 1), Wp, bp)

    LB = L * B
    info = plsc.get_sparse_core_info()
    nw = info.num_cores * info.num_subcores
    idxT = jnp.swapaxes(indices, 0, 1)
    idx2 = idxT.reshape(nw, LB // nw // 128, 128).astype(jnp.int32)
    gather_k = _make_gather(LB, info.num_cores, info.num_subcores)
    xb = gather_k(G, idx2)

    xb3 = xb.reshape(L, B, HP)
    mask3 = (indices != 0).astype(jnp.int32).reshape(B, L, 1)
    probs = _rnn_scan(xb3, mask3, Up, Wfcp, bfcp)
    return probs[:, :C]


# speedup vs baseline: 2.1486x; 1.5588x over previous
import functools

import jax
import jax.numpy as jnp
from jax import lax
from jax.experimental import pallas as pl
from jax.experimental.pallas import tpu as pltpu
from jax.experimental.pallas import tpu_sc as plsc

HP = 128
HPK = 64
CP = 64


def _gw_body(tt_ref, w_ref, b_ref, o_ref):
    o_ref[...] = (
        lax.dot_general(
            tt_ref[...], w_ref[...],
            dimension_numbers=(((0,), (0,)), ((), ())),
            preferred_element_type=jnp.float32,
        )
        + b_ref[...]
    )


def _table_times_w(tableT, Wp, bp, vblk=2048):
    E, V = tableT.shape
    return pl.pallas_call(
        _gw_body,
        grid=(pl.cdiv(V, vblk),),
        in_specs=[
            pl.BlockSpec((E, vblk), lambda i: (0, i)),
            pl.BlockSpec((E, HP), lambda i: (0, 0)),
            pl.BlockSpec((1, HP), lambda i: (0, 0)),
        ],
        out_specs=pl.BlockSpec((vblk, HP), lambda i: (i, 0)),
        out_shape=jax.ShapeDtypeStruct((V, HP), jnp.float32),
    )(tableT, Wp, bp)


def _make_gather(LB, nc, ns, k_grp=5):
    nw = nc * ns
    per_w = LB // nw
    grp = k_grp * 128
    n_groups = per_w // grp
    assert per_w % grp == 0

    mesh = plsc.VectorSubcoreMesh(core_axis_name="c", subcore_axis_name="s")

    @functools.partial(
        pl.kernel,
        mesh=mesh,
        out_type=jax.ShapeDtypeStruct((LB, HP), jnp.float32),
        scratch_types=[
            pltpu.VMEM((per_w // 128, 128), jnp.int32),
            pltpu.VMEM((grp, HP), jnp.float32),
            pltpu.SemaphoreType.DMA,
        ],
    )
    def gather_k(g_hbm, idx_hbm, out_hbm, idx_v, rows_v, sem):
        wid = lax.axis_index("s") * nc + lax.axis_index("c")
        base = wid * per_w
        pltpu.sync_copy(idx_hbm.at[wid], idx_v)
        for g in range(n_groups):
            off = base + g * grp
            descs = [
                pltpu.async_copy(
                    g_hbm.at[idx_v.at[g * k_grp + j]],
                    rows_v.at[pl.ds(j * 128, 128)],
                    sem,
                )
                for j in range(k_grp)
            ]
            for d in descs:
                d.wait()
            pltpu.sync_copy(rows_v, out_hbm.at[pl.ds(off, grp)])

    return gather_k


def _scan_body(n_steps, t_blk, xb_ref, idx_ref, u_ref, wfc_ref, bfc_ref,
               o_ref, h_ref):
    step = pl.program_id(0)

    @pl.when(step == 0)
    def _init():
        h_ref[...] = jnp.zeros_like(h_ref)

    HB = xb_ref.shape[1] // 2
    ha = h_ref[: HB, :]
    hb = h_ref[HB:, :]
    u = u_ref[...]
    for t in range(t_blk):
        xa = xb_ref[t, :HB, :]
        xb = xb_ref[t, HB:, :]
        ma = idx_ref[t, :HB, :] != 0
        mb = idx_ref[t, HB:, :] != 0
        hna = jnp.tanh(xa + jnp.dot(ha, u, preferred_element_type=jnp.float32))
        hnb = jnp.tanh(xb + jnp.dot(hb, u, preferred_element_type=jnp.float32))
        ha = jnp.where(ma, hna, ha)
        hb = jnp.where(mb, hnb, hb)
    h_ref[: HB, :] = ha
    h_ref[HB:, :] = hb

    @pl.when(step == n_steps - 1)
    def _head():
        h = jnp.concatenate([ha, hb], axis=0)
        logits = (
            jnp.dot(h, wfc_ref[...], preferred_element_type=jnp.float32)
            + bfc_ref[...]
        )
        mx = jnp.max(logits, axis=-1, keepdims=True)
        e = jnp.exp(logits - mx)
        o_ref[...] = e / jnp.sum(e, axis=-1, keepdims=True)


def _rnn_scan(xb3, idx3, Up, Wfcp, bfcp, t_blk=8):
    L, B, _ = xb3.shape
    n_steps = L // t_blk
    return pl.pallas_call(
        functools.partial(_scan_body, n_steps, t_blk),
        grid=(n_steps,),
        in_specs=[
            pl.BlockSpec((t_blk, B, HP), lambda i: (i, 0, 0)),
            pl.BlockSpec((t_blk, B, 1), lambda i: (i, 0, 0)),
            pl.BlockSpec((HP, HP), lambda i: (0, 0)),
            pl.BlockSpec((HP, CP), lambda i: (0, 0)),
            pl.BlockSpec((1, CP), lambda i: (0, 0)),
        ],
        out_specs=pl.BlockSpec((B, CP), lambda i: (0, 0)),
        out_shape=jax.ShapeDtypeStruct((B, CP), jnp.float32),
        scratch_shapes=[pltpu.VMEM((B, HP), jnp.float32)],
    )(xb3, idx3, Up, Wfcp, bfcp)


def kernel(indices, table, W, U, b, Wfc, bfc):
    B, L = indices.shape
    V, E = table.shape
    H = W.shape[1]
    C = Wfc.shape[1]

    Wp = jnp.pad(W, ((0, 0), (0, HP - H)))
    bp = jnp.pad(b, (0, HP - H)).reshape(1, HP)
    Up = jnp.pad(U, ((0, HP - H), (0, HP - H)))
    Wfcp = jnp.pad(Wfc, ((0, HP - H), (0, CP - C)))
    bfcp = jnp.pad(bfc, (0, CP - C), constant_values=-1e30).reshape(1, CP)

    G = _table_times_w(jnp.swapaxes(table, 0, 1), Wp, bp)

    LB = L * B
    info = plsc.get_sparse_core_info()
    nw = info.num_cores * info.num_subcores
    idxT = jnp.swapaxes(indices, 0, 1)
    idx2 = idxT.reshape(nw, LB // nw // 128, 128).astype(jnp.int32)
    gather_k = _make_gather(LB, info.num_cores, info.num_subcores)
    xb = gather_k(G, idx2)

    xb3 = xb.reshape(L, B, HP)
    mask3 = idxT.reshape(L, B, 1)
    probs = _rnn_scan(xb3, mask3, Up, Wfcp, bfcp)
    return probs[:, :C]


# speedup vs baseline: 2.1790x; 1.0141x over previous
import functools

import jax
import jax.numpy as jnp
from jax import lax
from jax.experimental import pallas as pl
from jax.experimental.pallas import tpu as pltpu
from jax.experimental.pallas import tpu_sc as plsc

HP = 128
HPK = 64
CP = 64


def _gw_body(tt_ref, w_ref, b_ref, o_ref):
    o_ref[...] = (
        lax.dot_general(
            tt_ref[...], w_ref[...],
            dimension_numbers=(((0,), (0,)), ((), ())),
            preferred_element_type=jnp.float32,
        )
        + b_ref[...]
    )


def _table_times_w(tableT, Wp, bp, vblk=2048):
    E, V = tableT.shape
    return pl.pallas_call(
        _gw_body,
        grid=(pl.cdiv(V, vblk),),
        in_specs=[
            pl.BlockSpec((E, vblk), lambda i: (0, i)),
            pl.BlockSpec((E, HP), lambda i: (0, 0)),
            pl.BlockSpec((1, HP), lambda i: (0, 0)),
        ],
        out_specs=pl.BlockSpec((vblk, HP), lambda i: (i, 0)),
        out_shape=jax.ShapeDtypeStruct((V, HP), jnp.float32),
    )(tableT, Wp, bp)


def _make_gather(LB, nc, ns, k_grp=5):
    nw = nc * ns
    per_w = LB // nw
    grp = k_grp * 128
    n_groups = per_w // grp
    assert per_w % grp == 0

    mesh = plsc.VectorSubcoreMesh(core_axis_name="c", subcore_axis_name="s")

    @functools.partial(
        pl.kernel,
        mesh=mesh,
        out_type=jax.ShapeDtypeStruct((LB, HP), jnp.float32),
        scratch_types=[
            pltpu.VMEM((per_w // 128, 128), jnp.int32),
            pltpu.VMEM((grp, HP), jnp.float32),
            pltpu.SemaphoreType.DMA,
        ],
    )
    def gather_k(g_hbm, idx_hbm, out_hbm, idx_v, rows_v, sem):
        wid = lax.axis_index("s") * nc + lax.axis_index("c")
        base = wid * per_w
        pltpu.sync_copy(idx_hbm.at[wid], idx_v)
        for g in range(n_groups):
            off = base + g * grp
            descs = [
                pltpu.async_copy(
                    g_hbm.at[idx_v.at[g * k_grp + j]],
                    rows_v.at[pl.ds(j * 128, 128)],
                    sem,
                )
                for j in range(k_grp)
            ]
            for d in descs:
                d.wait()
            pltpu.sync_copy(rows_v, out_hbm.at[pl.ds(off, grp)])

    return gather_k


def _scan_body(n_steps, t_blk, emit_probs, xb_ref, idx_ref, u_ref, wfc_ref,
               bfc_ref, hin_ref, o_ref, h_ref):
    step = pl.program_id(0)

    @pl.when(step == 0)
    def _init():
        h_ref[...] = hin_ref[...]

    HB = xb_ref.shape[1] // 2
    ha = h_ref[: HB, :]
    hb = h_ref[HB:, :]
    u = u_ref[...]
    for t in range(t_blk):
        xa = xb_ref[t, :HB, :]
        xb = xb_ref[t, HB:, :]
        ma = idx_ref[t, :HB, :] != 0
        mb = idx_ref[t, HB:, :] != 0
        hna = jnp.tanh(xa + jnp.dot(ha, u, preferred_element_type=jnp.float32))
        hnb = jnp.tanh(xb + jnp.dot(hb, u, preferred_element_type=jnp.float32))
        ha = jnp.where(ma, hna, ha)
        hb = jnp.where(mb, hnb, hb)
    h_ref[: HB, :] = ha
    h_ref[HB:, :] = hb

    @pl.when(step == n_steps - 1)
    def _tail():
        h = jnp.concatenate([ha, hb], axis=0)
        if emit_probs:
            logits = (
                jnp.dot(h, wfc_ref[...], preferred_element_type=jnp.float32)
                + bfc_ref[...]
            )
            mx = jnp.max(logits, axis=-1, keepdims=True)
            e = jnp.exp(logits - mx)
            o_ref[...] = e / jnp.sum(e, axis=-1, keepdims=True)
        else:
            o_ref[...] = h


def _rnn_scan(xb3, idx3, Up, Wfcp, bfcp, h0, emit_probs, t_blk=8):
    Lc, B, _ = xb3.shape
    n_steps = Lc // t_blk
    oc = CP if emit_probs else HP
    return pl.pallas_call(
        functools.partial(_scan_body, n_steps, t_blk, emit_probs),
        grid=(n_steps,),
        in_specs=[
            pl.BlockSpec((t_blk, B, HP), lambda i: (i, 0, 0)),
            pl.BlockSpec((t_blk, B, 1), lambda i: (i, 0, 0)),
            pl.BlockSpec((HP, HP), lambda i: (0, 0)),
            pl.BlockSpec((HP, CP), lambda i: (0, 0)),
            pl.BlockSpec((1, CP), lambda i: (0, 0)),
            pl.BlockSpec((B, HP), lambda i: (0, 0)),
        ],
        out_specs=pl.BlockSpec((B, oc), lambda i: (0, 0)),
        out_shape=jax.ShapeDtypeStruct((B, oc), jnp.float32),
        scratch_shapes=[pltpu.VMEM((B, HP), jnp.float32)],
    )(xb3, idx3, Up, Wfcp, bfcp, h0)


def kernel(indices, table, W, U, b, Wfc, bfc):
    B, L = indices.shape
    V, E = table.shape
    H = W.shape[1]
    C = Wfc.shape[1]

    Wp = jnp.pad(W, ((0, 0), (0, HP - H)))
    bp = jnp.pad(b, (0, HP - H)).reshape(1, HP)
    Up = jnp.pad(U, ((0, HP - H), (0, HP - H)))
    Wfcp = jnp.pad(Wfc, ((0, HP - H), (0, CP - C)))
    bfcp = jnp.pad(bfc, (0, CP - C), constant_values=-1e30).reshape(1, CP)

    G = _table_times_w(jnp.swapaxes(table, 0, 1), Wp, bp)

    info = plsc.get_sparse_core_info()
    nw = info.num_cores * info.num_subcores
    idxT = jnp.swapaxes(indices, 0, 1)

    Lh = L // 2
    LBh = Lh * B
    gather_k = _make_gather(LBh, info.num_cores, info.num_subcores)
    idx2a = idxT[:Lh].reshape(nw, LBh // nw // 128, 128).astype(jnp.int32)
    idx2b = idxT[Lh:].reshape(nw, LBh // nw // 128, 128).astype(jnp.int32)
    xb_a = gather_k(G, idx2a).reshape(Lh, B, HP)
    xb_b = gather_k(G, idx2b).reshape(Lh, B, HP)

    mask_a = idxT[:Lh].reshape(Lh, B, 1)
    mask_b = idxT[Lh:].reshape(Lh, B, 1)
    h0 = jnp.zeros((B, HP), jnp.float32)
    hmid = _rnn_scan(xb_a, mask_a, Up, Wfcp, bfcp, h0, False)
    probs = _rnn_scan(xb_b, mask_b, Up, Wfcp, bfcp, hmid, True)
    return probs[:, :C]
